# baseline (speedup 1.0000x reference)
import functools

import jax
import jax.numpy as jnp
from jax import lax
from jax.experimental import pallas as pl
from jax.experimental.pallas import tpu as pltpu
from jax.experimental.pallas import tpu_sc as plsc

NC, NS, LANES = 2, 16, 16
NW = NC * NS
CHUNK = 256
N_PAD = 10240
ROWS_PER_TILE = N_PAD // NS
CPW = 40
CPW0 = 72
CPW1 = 8
NCH = NS * (CPW0 + CPW1)
EPI_ROWS = 160
IDXH = 36
NBUF = 3
E_PAD = NCH * CHUNK
R_BLK = 10240
GRID = N_PAD // R_BLK


def _mesh():
  return plsc.VectorSubcoreMesh(
      core_axis_name="c", subcore_axis_name="s",
      num_cores=NC, num_subcores=NS)


@functools.lru_cache(maxsize=None)
def _make_sc_scatter(h):
  def body(src_hbm, dst_hbm, g_hbm, out0, out1, src_v, dst_v, rows0, rows1,
           rows2, ebuf, accum, gsem0, gsem1, gsem2, ssem0, ssem1, ssem2):
    c = lax.axis_index("c")
    s = lax.axis_index("s")

    def zero_row(i, carry):
      for jj in range(h // LANES):
        ebuf[i, pl.ds(jj * LANES, LANES)] = jnp.zeros((LANES,), jnp.float32)
      return carry

    lax.fori_loop(0, EPI_ROWS, zero_row, 0)
    for part in range(ROWS_PER_TILE // EPI_ROWS):
      pltpu.sync_copy(
          ebuf, accum.at[pl.ds(s * ROWS_PER_TILE + part * EPI_ROWS, EPI_ROWS)])
    plsc.subcore_barrier()

    def gstart(j, buf, sem):
      pltpu.async_copy(g_hbm.at[src_v.at[j]], buf, sem)

    def gwait(j, buf, sem):
      pltpu.make_async_copy(g_hbm.at[src_v.at[j]], buf, sem).wait()

    def sstart(j, buf, sem):
      pltpu.async_copy(buf, accum.at[dst_v.at[j]], sem, add=True)

    def swait(j, buf, sem):
      pltpu.make_async_copy(buf, accum.at[dst_v.at[j]], sem).wait()

    bufs = (rows0, rows1, rows2)
    gsems = (gsem0, gsem1, gsem2)
    ssems = (ssem0, ssem1, ssem2)

    def run_ring(cnt):
      steps = cnt // NBUF
      for b in range(NBUF):
        gstart(b, bufs[b], gsems[b])

      def step(t, carry):
        j = NBUF * t
        for b in range(NBUF):
          gwait(j + b, bufs[b], gsems[b])
          sstart(j + b, bufs[b], ssems[b])
        for b in range(NBUF):
          swait(j + b, bufs[b], ssems[b])

          @pl.when(t < steps - 1)
          def _():
            gstart(j + b + NBUF, bufs[b], gsems[b])

        return carry

      lax.fori_loop(0, steps, step, 0)

    def run_pairs(cnt):
      steps = cnt // 2
      gstart(0, rows0, gsem0)
      gstart(1, rows1, gsem1)

      def step(t, carry):
        j0 = 2 * t
        j1 = j0 + 1
        gwait(j0, rows0, gsem0)
        sstart(j0, rows0, ssem0)
        gwait(j1, rows1, gsem1)
        sstart(j1, rows1, ssem1)
        swait(j0, rows0, ssem0)
        swait(j1, rows1, ssem1)

        @pl.when(t < steps - 1)
        def _():
          gstart(j0 + 2, rows0, gsem0)
          gstart(j1 + 2, rows1, gsem1)

        return carry

      lax.fori_loop(0, steps, step, 0)

    def run_edges(base, cpw):
      for h0 in range(0, cpw, IDXH):
        cnt = min(IDXH, cpw - h0)
        pltpu.sync_copy(src_hbm.at[pl.ds(base + h0, cnt)],
                        src_v.at[pl.ds(0, cnt)])
        pltpu.sync_copy(dst_hbm.at[pl.ds(base + h0, cnt)],
                        dst_v.at[pl.ds(0, cnt)])
        if cnt % NBUF == 0:
          run_ring(cnt)
        else:
          run_pairs(cnt)

    @pl.when(c == 0)
    def _():
      run_edges(s * CPW0, CPW0)

    if CPW1:
      @pl.when(c == 1)
      def _():
        run_edges(NS * CPW0 + s * CPW1, CPW1)

    plsc.subcore_barrier()

    for part in range(ROWS_PER_TILE // EPI_ROWS):
      psl = pl.ds(s * ROWS_PER_TILE + part * EPI_ROWS, EPI_ROWS)
      pltpu.sync_copy(accum.at[psl], ebuf)

      @pl.when(c == 0)
      def _():
        pltpu.sync_copy(ebuf, out0.at[psl])

      @pl.when(c == 1)
      def _():
        pltpu.sync_copy(ebuf, out1.at[psl])

  return pl.kernel(
      body,
      out_type=[jax.ShapeDtypeStruct((N_PAD, h), jnp.float32)] * 2,
      mesh=_mesh(),
      scratch_types=[
          pltpu.VMEM((IDXH, CHUNK), jnp.int32),
          pltpu.VMEM((IDXH, CHUNK), jnp.int32),
          pltpu.VMEM((CHUNK, h), jnp.float32),
          pltpu.VMEM((CHUNK, h), jnp.float32),
          pltpu.VMEM((CHUNK, h), jnp.float32),
          pltpu.VMEM((EPI_ROWS, h), jnp.float32),
          pltpu.VMEM_SHARED((N_PAD, h), jnp.float32),
          pltpu.SemaphoreType.DMA,
          pltpu.SemaphoreType.DMA,
          pltpu.SemaphoreType.DMA,
          pltpu.SemaphoreType.DMA,
          pltpu.SemaphoreType.DMA,
          pltpu.SemaphoreType.DMA,
      ],
      compiler_params=pltpu.CompilerParams(use_tc_tiling_on_sc=False),
  )


@functools.lru_cache(maxsize=None)
def _make_sc_degree():
  h = LANES

  def body(dst_hbm, out0, out1, dst_v, ones_v, ebuf, accum):
    c = lax.axis_index("c")
    s = lax.axis_index("s")
    w = c * NS + s
    pltpu.sync_copy(dst_hbm.at[pl.ds(w * CPW, CPW)], dst_v)

    def fill_ones(i, carry):
      ones_v[i, :] = jnp.ones((LANES,), jnp.float32)
      return carry

    lax.fori_loop(0, CHUNK, fill_ones, 0)

    def zero_row(i, carry):
      ebuf[i, :] = jnp.zeros((LANES,), jnp.float32)
      return carry

    lax.fori_loop(0, ROWS_PER_TILE, zero_row, 0)
    sl = pl.ds(s * ROWS_PER_TILE, ROWS_PER_TILE)
    pltpu.sync_copy(ebuf, accum.at[sl])
    plsc.subcore_barrier()

    def step(j, carry):
      pltpu.sync_copy(ones_v, accum.at[dst_v.at[j]], add=True)
      return carry

    lax.fori_loop(0, CPW, step, 0)
    plsc.subcore_barrier()

    pltpu.sync_copy(accum.at[sl], ebuf)

    @pl.when(c == 0)
    def _():
      pltpu.sync_copy(ebuf, out0.at[sl])

    @pl.when(c == 1)
    def _():
      pltpu.sync_copy(ebuf, out1.at[sl])

  return pl.kernel(
      body,
      out_type=[jax.ShapeDtypeStruct((N_PAD, h), jnp.float32)] * 2,
      mesh=_mesh(),
      scratch_types=[
          pltpu.VMEM((CPW, CHUNK), jnp.int32),
          pltpu.VMEM((CHUNK, h), jnp.float32),
          pltpu.VMEM((ROWS_PER_TILE, h), jnp.float32),
          pltpu.VMEM_SHARED((N_PAD, h), jnp.float32),
      ],
      compiler_params=pltpu.CompilerParams(use_tc_tiling_on_sc=False),
  )


def _tc_prep(deg0, deg1, xp, W1):
  d = xp.shape[1]
  hh = W1.shape[1]

  def body(d0, d1, x_r, w_r, g_r, dinv_r):
    deg = d0[:, :1] + d1[:, :1] + 1.0
    dinv = jnp.where(deg > 0, lax.rsqrt(deg), 0.0)
    m = jnp.dot(x_r[...], w_r[...], preferred_element_type=jnp.float32)
    g_r[...] = m * dinv
    dinv_r[...] = dinv

  return pl.pallas_call(
      body,
      grid=(GRID,),
      in_specs=[
          pl.BlockSpec((R_BLK, LANES), lambda i: (i, 0)),
          pl.BlockSpec((R_BLK, LANES), lambda i: (i, 0)),
          pl.BlockSpec((R_BLK, d), lambda i: (i, 0)),
          pl.BlockSpec((d, hh), lambda i: (0, 0)),
      ],
      out_specs=[
          pl.BlockSpec((R_BLK, hh), lambda i: (i, 0)),
          pl.BlockSpec((R_BLK, 1), lambda i: (i, 0)),
      ],
      out_shape=[
          jax.ShapeDtypeStruct((N_PAD, hh), jnp.float32),
          jax.ShapeDtypeStruct((N_PAD, 1), jnp.float32),
      ],
  )(deg0, deg1, xp, W1)


def _tc_layer(p0, p1, g, dinv, b, W):
  h_in = g.shape[1]
  h_out = W.shape[1]

  def body(p0r, p1r, gr, dr, br, wr, outr):
    total = p0r[...] + p1r[...] + gr[...]
    dv = dr[...]
    z = jnp.maximum(dv * total + br[...], 0.0)
    outr[...] = jnp.dot(dv * z, wr[...], preferred_element_type=jnp.float32)

  return pl.pallas_call(
      body,
      grid=(GRID,),
      in_specs=[
          pl.BlockSpec((R_BLK, h_in), lambda i: (i, 0)),
          pl.BlockSpec((R_BLK, h_in), lambda i: (i, 0)),
          pl.BlockSpec((R_BLK, h_in), lambda i: (i, 0)),
          pl.BlockSpec((R_BLK, 1), lambda i: (i, 0)),
          pl.BlockSpec((1, h_in), lambda i: (0, 0)),
          pl.BlockSpec((h_in, h_out), lambda i: (0, 0)),
      ],
      out_specs=pl.BlockSpec((R_BLK, h_out), lambda i: (i, 0)),
      out_shape=jax.ShapeDtypeStruct((N_PAD, h_out), jnp.float32),
  )(p0, p1, g, dinv, b, W)


def _tc_final(p0, p1, g, dinv, b, d_out):
  h_in = g.shape[1]

  def body(p0r, p1r, gr, dr, br, outr):
    total = p0r[...] + p1r[...] + gr[...]
    outr[...] = (dr[...] * total)[:, :d_out] + br[...]

  return pl.pallas_call(
      body,
      grid=(GRID,),
      in_specs=[
          pl.BlockSpec((R_BLK, h_in), lambda i: (i, 0)),
          pl.BlockSpec((R_BLK, h_in), lambda i: (i, 0)),
          pl.BlockSpec((R_BLK, h_in), lambda i: (i, 0)),
          pl.BlockSpec((R_BLK, 1), lambda i: (i, 0)),
          pl.BlockSpec((1, d_out), lambda i: (0, 0)),
      ],
      out_specs=pl.BlockSpec((R_BLK, d_out), lambda i: (i, 0)),
      out_shape=jax.ShapeDtypeStruct((N_PAD, d_out), jnp.float32),
  )(p0, p1, g, dinv, b)


def kernel(x, edge_index, W1, b1, W2, b2, W3, b3):
  n = x.shape[0]
  e = edge_index.shape[1]
  d_out = W3.shape[1]

  xp = jnp.zeros((N_PAD, x.shape[1]), jnp.float32).at[:n].set(x)
  fill = jnp.full((E_PAD - e,), n, jnp.int32)
  src2 = jnp.concatenate([edge_index[0].astype(jnp.int32), fill])
  src2 = src2.reshape(NW * CPW, CHUNK)
  dst2 = jnp.concatenate([edge_index[1].astype(jnp.int32), fill])
  dst2 = dst2.reshape(NW * CPW, CHUNK)

  deg0, deg1 = _make_sc_degree()(dst2)
  g1, dinv = _tc_prep(deg0, deg1, xp, W1)

  p10, p11 = _make_sc_scatter(64)(src2, dst2, g1)
  g2 = _tc_layer(p10, p11, g1, dinv, b1.reshape(1, -1), W2)

  p20, p21 = _make_sc_scatter(32)(src2, dst2, g2)
  W3p = jnp.zeros((W3.shape[0], LANES), jnp.float32).at[:, :d_out].set(W3)
  g3 = _tc_layer(p20, p21, g2, dinv, b2.reshape(1, -1), W3p)

  p30, p31 = _make_sc_scatter(LANES)(src2, dst2, g3)
  out = _tc_final(p30, p31, g3, dinv, b3.reshape(1, -1), d_out)
  return out[:n]


# speedup vs baseline: 1.0220x; 1.0220x over previous
import functools

import jax
import jax.numpy as jnp
from jax import lax
from jax.experimental import pallas as pl
from jax.experimental.pallas import tpu as pltpu
from jax.experimental.pallas import tpu_sc as plsc

NC, NS, LANES = 2, 16, 16
NW = NC * NS
CHUNK = 256
N_PAD = 10240
ROWS_PER_TILE = N_PAD // NS
CPW = 40
CPW0 = 74
CPW1 = 6
NCH = NS * (CPW0 + CPW1)
EPI_ROWS = 160
IDXH = 36
NBUF = 3
E_PAD = NCH * CHUNK
R_BLK = 5120
GRID = N_PAD // R_BLK


def _mesh():
  return plsc.VectorSubcoreMesh(
      core_axis_name="c", subcore_axis_name="s",
      num_cores=NC, num_subcores=NS)


@functools.lru_cache(maxsize=None)
def _make_sc_scatter(h):
  def body(src_hbm, dst_hbm, g_hbm, out0, out1, src_v, dst_v, rows0, rows1,
           rows2, ebuf, accum, gsem0, gsem1, gsem2, ssem0, ssem1, ssem2):
    c = lax.axis_index("c")
    s = lax.axis_index("s")

    def zero_row(i, carry):
      for jj in range(h // LANES):
        ebuf[i, pl.ds(jj * LANES, LANES)] = jnp.zeros((LANES,), jnp.float32)
      return carry

    lax.fori_loop(0, EPI_ROWS, zero_row, 0)
    for part in range(ROWS_PER_TILE // EPI_ROWS):
      pltpu.sync_copy(
          ebuf, accum.at[pl.ds(s * ROWS_PER_TILE + part * EPI_ROWS, EPI_ROWS)])
    plsc.subcore_barrier()

    def gstart(j, buf, sem):
      pltpu.async_copy(g_hbm.at[src_v.at[j]], buf, sem)

    def gwait(j, buf, sem):
      pltpu.make_async_copy(g_hbm.at[src_v.at[j]], buf, sem).wait()

    def sstart(j, buf, sem):
      pltpu.async_copy(buf, accum.at[dst_v.at[j]], sem, add=True)

    def swait(j, buf, sem):
      pltpu.make_async_copy(buf, accum.at[dst_v.at[j]], sem).wait()

    bufs = (rows0, rows1, rows2)
    gsems = (gsem0, gsem1, gsem2)
    ssems = (ssem0, ssem1, ssem2)

    def run_ring(cnt):
      steps = cnt // NBUF
      for b in range(NBUF):
        gstart(b, bufs[b], gsems[b])

      def step(t, carry):
        j = NBUF * t
        for b in range(NBUF):
          gwait(j + b, bufs[b], gsems[b])
          sstart(j + b, bufs[b], ssems[b])
        for b in range(NBUF):
          swait(j + b, bufs[b], ssems[b])

          @pl.when(t < steps - 1)
          def _():
            gstart(j + b + NBUF, bufs[b], gsems[b])

        return carry

      lax.fori_loop(0, steps, step, 0)

    def run_pairs(cnt):
      steps = cnt // 2
      gstart(0, rows0, gsem0)
      gstart(1, rows1, gsem1)

      def step(t, carry):
        j0 = 2 * t
        j1 = j0 + 1
        gwait(j0, rows0, gsem0)
        sstart(j0, rows0, ssem0)
        gwait(j1, rows1, gsem1)
        sstart(j1, rows1, ssem1)
        swait(j0, rows0, ssem0)
        swait(j1, rows1, ssem1)

        @pl.when(t < steps - 1)
        def _():
          gstart(j0 + 2, rows0, gsem0)
          gstart(j1 + 2, rows1, gsem1)

        return carry

      lax.fori_loop(0, steps, step, 0)

    def run_edges(base, cpw):
      for h0 in range(0, cpw, IDXH):
        cnt = min(IDXH, cpw - h0)
        pltpu.sync_copy(src_hbm.at[pl.ds(base + h0, cnt)],
                        src_v.at[pl.ds(0, cnt)])
        pltpu.sync_copy(dst_hbm.at[pl.ds(base + h0, cnt)],
                        dst_v.at[pl.ds(0, cnt)])
        if cnt % NBUF == 0:
          run_ring(cnt)
        else:
          run_pairs(cnt)

    @pl.when(c == 0)
    def _():
      run_edges(s * CPW0, CPW0)

    if CPW1:
      @pl.when(c == 1)
      def _():
        run_edges(NS * CPW0 + s * CPW1, CPW1)

    plsc.subcore_barrier()

    for part in range(ROWS_PER_TILE // EPI_ROWS):
      psl = pl.ds(s * ROWS_PER_TILE + part * EPI_ROWS, EPI_ROWS)
      pltpu.sync_copy(accum.at[psl], ebuf)

      @pl.when(c == 0)
      def _():
        pltpu.sync_copy(ebuf, out0.at[psl])

      @pl.when(c == 1)
      def _():
        pltpu.sync_copy(ebuf, out1.at[psl])

  return pl.kernel(
      body,
      out_type=[jax.ShapeDtypeStruct((N_PAD, h), jnp.float32)] * 2,
      mesh=_mesh(),
      scratch_types=[
          pltpu.VMEM((IDXH, CHUNK), jnp.int32),
          pltpu.VMEM((IDXH, CHUNK), jnp.int32),
          pltpu.VMEM((CHUNK, h), jnp.float32),
          pltpu.VMEM((CHUNK, h), jnp.float32),
          pltpu.VMEM((CHUNK, h), jnp.float32),
          pltpu.VMEM((EPI_ROWS, h), jnp.float32),
          pltpu.VMEM_SHARED((N_PAD, h), jnp.float32),
          pltpu.SemaphoreType.DMA,
          pltpu.SemaphoreType.DMA,
          pltpu.SemaphoreType.DMA,
          pltpu.SemaphoreType.DMA,
          pltpu.SemaphoreType.DMA,
          pltpu.SemaphoreType.DMA,
      ],
      compiler_params=pltpu.CompilerParams(use_tc_tiling_on_sc=False),
  )


@functools.lru_cache(maxsize=None)
def _make_sc_degree():
  h = LANES

  def body(dst_hbm, out0, out1, dst_v, ones_v, ebuf, accum):
    c = lax.axis_index("c")
    s = lax.axis_index("s")
    w = c * NS + s
    pltpu.sync_copy(dst_hbm.at[pl.ds(w * CPW, CPW)], dst_v)

    def fill_ones(i, carry):
      ones_v[i, :] = jnp.ones((LANES,), jnp.float32)
      return carry

    lax.fori_loop(0, CHUNK, fill_ones, 0)

    def zero_row(i, carry):
      ebuf[i, :] = jnp.zeros((LANES,), jnp.float32)
      return carry

    lax.fori_loop(0, ROWS_PER_TILE, zero_row, 0)
    sl = pl.ds(s * ROWS_PER_TILE, ROWS_PER_TILE)
    pltpu.sync_copy(ebuf, accum.at[sl])
    plsc.subcore_barrier()

    def step(j, carry):
      pltpu.sync_copy(ones_v, accum.at[dst_v.at[j]], add=True)
      return carry

    lax.fori_loop(0, CPW, step, 0)
    plsc.subcore_barrier()

    pltpu.sync_copy(accum.at[sl], ebuf)

    @pl.when(c == 0)
    def _():
      pltpu.sync_copy(ebuf, out0.at[sl])

    @pl.when(c == 1)
    def _():
      pltpu.sync_copy(ebuf, out1.at[sl])

  return pl.kernel(
      body,
      out_type=[jax.ShapeDtypeStruct((N_PAD, h), jnp.float32)] * 2,
      mesh=_mesh(),
      scratch_types=[
          pltpu.VMEM((CPW, CHUNK), jnp.int32),
          pltpu.VMEM((CHUNK, h), jnp.float32),
          pltpu.VMEM((ROWS_PER_TILE, h), jnp.float32),
          pltpu.VMEM_SHARED((N_PAD, h), jnp.float32),
      ],
      compiler_params=pltpu.CompilerParams(use_tc_tiling_on_sc=False),
  )


def _tc_prep(deg0, deg1, xp, W1):
  d = xp.shape[1]
  hh = W1.shape[1]

  def body(d0, d1, x_r, w_r, g_r, dinv_r):
    deg = d0[:, :1] + d1[:, :1] + 1.0
    dinv = jnp.where(deg > 0, lax.rsqrt(deg), 0.0)
    m = jnp.dot(x_r[...], w_r[...], preferred_element_type=jnp.float32)
    g_r[...] = m * dinv
    dinv_r[...] = dinv

  return pl.pallas_call(
      body,
      grid=(GRID,),
      in_specs=[
          pl.BlockSpec((R_BLK, LANES), lambda i: (i, 0)),
          pl.BlockSpec((R_BLK, LANES), lambda i: (i, 0)),
          pl.BlockSpec((R_BLK, d), lambda i: (i, 0)),
          pl.BlockSpec((d, hh), lambda i: (0, 0)),
      ],
      out_specs=[
          pl.BlockSpec((R_BLK, hh), lambda i: (i, 0)),
          pl.BlockSpec((R_BLK, 1), lambda i: (i, 0)),
      ],
      out_shape=[
          jax.ShapeDtypeStruct((N_PAD, hh), jnp.float32),
          jax.ShapeDtypeStruct((N_PAD, 1), jnp.float32),
      ],
  )(deg0, deg1, xp, W1)


def _tc_layer(p0, p1, g, dinv, b, W):
  h_in = g.shape[1]
  h_out = W.shape[1]

  def body(p0r, p1r, gr, dr, br, wr, outr):
    total = p0r[...] + p1r[...] + gr[...]
    dv = dr[...]
    z = jnp.maximum(dv * total + br[...], 0.0)
    outr[...] = jnp.dot(dv * z, wr[...], preferred_element_type=jnp.float32)

  return pl.pallas_call(
      body,
      grid=(GRID,),
      in_specs=[
          pl.BlockSpec((R_BLK, h_in), lambda i: (i, 0)),
          pl.BlockSpec((R_BLK, h_in), lambda i: (i, 0)),
          pl.BlockSpec((R_BLK, h_in), lambda i: (i, 0)),
          pl.BlockSpec((R_BLK, 1), lambda i: (i, 0)),
          pl.BlockSpec((1, h_in), lambda i: (0, 0)),
          pl.BlockSpec((h_in, h_out), lambda i: (0, 0)),
      ],
      out_specs=pl.BlockSpec((R_BLK, h_out), lambda i: (i, 0)),
      out_shape=jax.ShapeDtypeStruct((N_PAD, h_out), jnp.float32),
  )(p0, p1, g, dinv, b, W)


def _tc_final(p0, p1, g, dinv, b, d_out):
  h_in = g.shape[1]

  def body(p0r, p1r, gr, dr, br, outr):
    total = p0r[...] + p1r[...] + gr[...]
    outr[...] = (dr[...] * total)[:, :d_out] + br[...]

  return pl.pallas_call(
      body,
      grid=(GRID,),
      in_specs=[
          pl.BlockSpec((R_BLK, h_in), lambda i: (i, 0)),
          pl.BlockSpec((R_BLK, h_in), lambda i: (i, 0)),
          pl.BlockSpec((R_BLK, h_in), lambda i: (i, 0)),
          pl.BlockSpec((R_BLK, 1), lambda i: (i, 0)),
          pl.BlockSpec((1, d_out), lambda i: (0, 0)),
      ],
      out_specs=pl.BlockSpec((R_BLK, d_out), lambda i: (i, 0)),
      out_shape=jax.ShapeDtypeStruct((N_PAD, d_out), jnp.float32),
  )(p0, p1, g, dinv, b)


def kernel(x, edge_index, W1, b1, W2, b2, W3, b3):
  n = x.shape[0]
  e = edge_index.shape[1]
  d_out = W3.shape[1]

  xp = jnp.zeros((N_PAD, x.shape[1]), jnp.float32).at[:n].set(x)
  fill = jnp.full((E_PAD - e,), n, jnp.int32)
  src2 = jnp.concatenate([edge_index[0].astype(jnp.int32), fill])
  src2 = src2.reshape(NW * CPW, CHUNK)
  dst2 = jnp.concatenate([edge_index[1].astype(jnp.int32), fill])
  dst2 = dst2.reshape(NW * CPW, CHUNK)

  deg0, deg1 = _make_sc_degree()(dst2)
  g1, dinv = _tc_prep(deg0, deg1, xp, W1)

  p10, p11 = _make_sc_scatter(64)(src2, dst2, g1)
  g2 = _tc_layer(p10, p11, g1, dinv, b1.reshape(1, -1), W2)

  p20, p21 = _make_sc_scatter(32)(src2, dst2, g2)
  W3p = jnp.zeros((W3.shape[0], LANES), jnp.float32).at[:, :d_out].set(W3)
  g3 = _tc_layer(p20, p21, g2, dinv, b2.reshape(1, -1), W3p)

  p30, p31 = _make_sc_scatter(LANES)(src2, dst2, g3)
  out = _tc_final(p30, p31, g3, dinv, b3.reshape(1, -1), d_out)
  return out[:n]
